# Initial kernel scaffold; baseline (speedup 1.0000x reference)
import jax
import jax.numpy as jnp
from jax.experimental import pallas as pl


def kernel(representations, targets_fine, label_map, tree_dist):
    raise NotImplementedError("write your pallas kernel here")


# speedup vs baseline: 9.8749x; 9.8749x over previous
import jax
import jax.numpy as jnp
from jax import lax
from jax.experimental import pallas as pl
from jax.experimental.pallas import tpu as pltpu
from jax.experimental.pallas import tpu_sc as plsc

NF = 100
NN = 120
NSEG = 128
B = 16384
D = 128
NCORES = 2
NSUB = 16
NW = NCORES * NSUB
ROWS = B // NW
CH = 128
NCHUNK = ROWS // CH


def _sc_segsum(reps_hbm, tgt_hbm, ones_hbm, zsum_hbm,
               sums_out, cnts_out,
               rows_v, tgt_v, ones_v, shared_sums, shared_cnts):
    cid = lax.axis_index("c")
    sid = lax.axis_index("s")
    wid = cid * NSUB + sid
    base = wid * ROWS

    pltpu.sync_copy(reps_hbm.at[pl.ds(base, ROWS)], rows_v)
    pltpu.sync_copy(tgt_hbm.at[pl.ds(wid * NCHUNK, NCHUNK)], tgt_v)
    pltpu.sync_copy(ones_hbm, ones_v)

    @pl.when(sid == 0)
    def _zero():
        pltpu.sync_copy(zsum_hbm, shared_sums)
        pltpu.sync_copy(zsum_hbm, shared_cnts)

    plsc.subcore_barrier()

    for j in range(NCHUNK):
        idx = tgt_v.at[j]
        pltpu.sync_copy(rows_v.at[pl.ds(j * CH, CH)], shared_sums.at[idx],
                        add=True)
        pltpu.sync_copy(ones_v, shared_cnts.at[idx], add=True)

    plsc.subcore_barrier()

    @pl.when(sid == 0)
    def _writeout():
        pltpu.sync_copy(shared_sums, sums_out.at[cid])
        pltpu.sync_copy(shared_cnts, cnts_out.at[cid])


import functools


@functools.lru_cache(maxsize=1)
def _make_seg_call():
    return pl.kernel(
        _sc_segsum,
        out_type=[
            jax.ShapeDtypeStruct((NCORES, NSEG, D), jnp.float32),
            jax.ShapeDtypeStruct((NCORES, NSEG, D), jnp.float32),
        ],
        mesh=plsc.VectorSubcoreMesh(core_axis_name="c", subcore_axis_name="s"),
        scratch_types=[
            pltpu.VMEM((ROWS, D), jnp.float32),
            pltpu.VMEM((NCHUNK, CH), jnp.int32),
            pltpu.VMEM((CH, D), jnp.float32),
            pltpu.VMEM_SHARED((NSEG, D), jnp.float32),
            pltpu.VMEM_SHARED((NSEG, D), jnp.float32),
        ],
    )


def _tc_tail(s_ref, c_ref, lm_ref, td_ref, o_ref):
    f32 = jnp.float32
    sums = s_ref[0] + s_ref[1]
    cnt = c_ref[0] + c_ref[1]
    cnt_col = cnt[:, 0:1]

    lm = lm_ref[0:1, :]
    ri = lax.broadcasted_iota(jnp.int32, (NSEG, NSEG), 0)
    ci = lax.broadcasted_iota(jnp.int32, (NSEG, NSEG), 1)
    eye = ri == ci

    a_fine = jnp.where(eye & (ri < NF), 1.0, 0.0)
    a_coarse = jnp.where((ri >= NF) & (ri < NN) & (lm == ri), 1.0, 0.0)
    amat = (a_fine + a_coarse).astype(f32)

    node_sums = jnp.dot(amat, sums, preferred_element_type=f32)
    cnt_row = jnp.sum(jnp.where(eye, cnt_col * jnp.ones((1, NSEG), f32), 0.0),
                      axis=0, keepdims=True)
    node_cnt = jnp.sum(amat * cnt_row, axis=1, keepdims=True)

    present = node_cnt > 0.0
    safe = jnp.where(present, node_cnt, 1.0)
    means = node_sums / safe

    gram = lax.dot_general(means, means, (((1,), (1,)), ((), ())),
                           preferred_element_type=f32)
    diag_col = jnp.sum(jnp.where(eye, gram, 0.0), axis=1, keepdims=True)
    diag_row = jnp.sum(jnp.where(eye, gram, 0.0), axis=0, keepdims=True)
    d2 = jnp.maximum(diag_col + diag_row - 2.0 * gram, 0.0)
    dist = jnp.sqrt(d2)

    node_cnt_row = jnp.sum(jnp.where(eye, node_cnt * jnp.ones((1, NSEG), f32),
                                     0.0), axis=0, keepdims=True)
    w = jnp.where((node_cnt > 0.0) & (node_cnt_row > 0.0)
                  & (ri < ci) & (ci < NN), 1.0, 0.0).astype(f32)

    y = td_ref[...]
    m = jnp.sum(w)
    mx = jnp.sum(w * dist) / m
    my = jnp.sum(w * y) / m
    xc = w * (dist - mx)
    yc = w * (y - my)
    num = jnp.sum(xc * yc)
    den = jnp.sqrt(jnp.sum(xc * xc)) * jnp.sqrt(jnp.sum(yc * yc))
    res = 1.0 - num / den
    res = jnp.where(jnp.isnan(res), jnp.array(1.0, f32), res)
    o_ref[...] = jnp.full((8, NSEG), res, f32)


_tail_call = pl.pallas_call(
    _tc_tail,
    out_shape=jax.ShapeDtypeStruct((8, NSEG), jnp.float32),
)


def kernel(representations, targets_fine, label_map, tree_dist):
    tgt2d = targets_fine.reshape(NW * NCHUNK, CH)
    ones_c = jnp.ones((CH, D), jnp.float32)
    zsum = jnp.zeros((NSEG, D), jnp.float32)
    lm_row = jnp.pad(label_map[:, 1], (0, NSEG - NF))
    lm_pad = jnp.broadcast_to(lm_row[None, :], (8, NSEG))
    td_pad = jnp.pad(tree_dist, ((0, NSEG - NN), (0, NSEG - NN)))

    sums2, cnts2 = _make_seg_call()(representations, tgt2d, ones_c, zsum)
    outb = _tail_call(sums2, cnts2, lm_pad, td_pad)
    return outb[0, 0]


# speedup vs baseline: 12.0955x; 1.2249x over previous
import functools

import jax
import jax.numpy as jnp
import numpy as np
from jax import lax
from jax.experimental import pallas as pl
from jax.experimental.pallas import tpu as pltpu
from jax.experimental.pallas import tpu_sc as plsc

NF = 100
NN = 120
NSEG = 128
B = 16384
D = 128
NCORES = 2
NSUB = 16
NW = NCORES * NSUB
ROWS = B // NW
CH = 128
NCHUNK = ROWS // CH


def _sc_segsum(reps_hbm, tgt_hbm, zsum_hbm, sums_out,
               rows_v, tgt_v, shared_sums):
    cid = lax.axis_index("c")
    sid = lax.axis_index("s")
    wid = cid * NSUB + sid
    base = wid * ROWS

    pltpu.sync_copy(reps_hbm.at[pl.ds(base, ROWS)], rows_v)
    pltpu.sync_copy(tgt_hbm.at[pl.ds(wid * NCHUNK, NCHUNK)], tgt_v)

    @pl.when(sid == 0)
    def _zero():
        pltpu.sync_copy(zsum_hbm, shared_sums)

    plsc.subcore_barrier()

    for j in range(NCHUNK):
        pltpu.sync_copy(rows_v.at[pl.ds(j * CH, CH)],
                        shared_sums.at[tgt_v.at[j]], add=True)

    plsc.subcore_barrier()

    @pl.when(sid == 0)
    def _writeout():
        pltpu.sync_copy(shared_sums, sums_out.at[cid])


@functools.lru_cache(maxsize=1)
def _make_seg_call():
    return pl.kernel(
        _sc_segsum,
        out_type=jax.ShapeDtypeStruct((NCORES, NSEG, D), jnp.float32),
        mesh=plsc.VectorSubcoreMesh(core_axis_name="c", subcore_axis_name="s"),
        scratch_types=[
            pltpu.VMEM((ROWS, D), jnp.float32),
            pltpu.VMEM((NCHUNK, CH), jnp.int32),
            pltpu.VMEM_SHARED((NSEG, D), jnp.float32),
        ],
    )


def _tc_counts(tgt_ref, o_ref):
    f32 = jnp.float32
    classrow = lax.broadcasted_iota(jnp.int32, (1, NSEG), 1)
    acc = jnp.zeros((1, NSEG), f32)
    for k in range(NW * NCHUNK):
        col = tgt_ref[:, k:k + 1]
        oh = jnp.where(col == classrow, 1.0, 0.0)
        acc = acc + jnp.sum(oh, axis=0, keepdims=True)
    o_ref[...] = jnp.broadcast_to(acc, (8, NSEG))


_counts_call = pl.pallas_call(
    _tc_counts,
    out_shape=jax.ShapeDtypeStruct((8, NSEG), jnp.float32),
)


def _tc_tail(s_ref, c_ref, lm_ref, td_ref, o_ref):
    f32 = jnp.float32
    sums = s_ref[0] + s_ref[1]
    cnt_row = c_ref[0:1, :]

    lm = lm_ref[0:1, :]
    ri = lax.broadcasted_iota(jnp.int32, (NSEG, NSEG), 0)
    ci = lax.broadcasted_iota(jnp.int32, (NSEG, NSEG), 1)
    eye = ri == ci

    a_fine = jnp.where(eye & (ri < NF), 1.0, 0.0)
    a_coarse = jnp.where((ri >= NF) & (ri < NN) & (lm == ri), 1.0, 0.0)
    amat = (a_fine + a_coarse).astype(f32)

    node_sums = jnp.dot(amat, sums, preferred_element_type=f32)
    node_cnt = jnp.sum(amat * cnt_row, axis=1, keepdims=True)

    present = node_cnt > 0.0
    safe = jnp.where(present, node_cnt, 1.0)
    means = node_sums / safe

    gram = lax.dot_general(means, means, (((1,), (1,)), ((), ())),
                           preferred_element_type=f32)
    diag_col = jnp.sum(jnp.where(eye, gram, 0.0), axis=1, keepdims=True)
    diag_row = jnp.sum(jnp.where(eye, gram, 0.0), axis=0, keepdims=True)
    d2 = jnp.maximum(diag_col + diag_row - 2.0 * gram, 0.0)
    dist = jnp.sqrt(d2)

    node_cnt_row = jnp.sum(jnp.where(eye, node_cnt * jnp.ones((1, NSEG), f32),
                                     0.0), axis=0, keepdims=True)
    w = jnp.where((node_cnt > 0.0) & (node_cnt_row > 0.0)
                  & (ri < ci) & (ci < NN), 1.0, 0.0).astype(f32)

    y = td_ref[...]
    m = jnp.sum(w)
    mx = jnp.sum(w * dist) / m
    my = jnp.sum(w * y) / m
    xc = w * (dist - mx)
    yc = w * (y - my)
    num = jnp.sum(xc * yc)
    den = jnp.sqrt(jnp.sum(xc * xc)) * jnp.sqrt(jnp.sum(yc * yc))
    res = 1.0 - num / den
    res = jnp.where(jnp.isnan(res), jnp.array(1.0, f32), res)
    o_ref[...] = jnp.full((8, NSEG), res, f32)


_tail_call = pl.pallas_call(
    _tc_tail,
    out_shape=jax.ShapeDtypeStruct((8, NSEG), jnp.float32),
)


def kernel(representations, targets_fine, label_map, tree_dist):
    tgt2d = targets_fine.reshape(NW * NCHUNK, CH)
    zsum = np.zeros((NSEG, D), np.float32)
    lm_row = jnp.pad(label_map[:, 1], (0, NSEG - NF))
    lm_pad = jnp.broadcast_to(lm_row[None, :], (8, NSEG))
    td_pad = jnp.pad(tree_dist, ((0, NSEG - NN), (0, NSEG - NN)))

    sums2 = _make_seg_call()(representations, tgt2d, zsum)
    cnts = _counts_call(tgt2d)
    outb = _tail_call(sums2, cnts, lm_pad, td_pad)
    return outb[0, 0]


# speedup vs baseline: 13.1082x; 1.0837x over previous
import functools

import jax
import jax.numpy as jnp
import numpy as np
from jax import lax
from jax.experimental import pallas as pl
from jax.experimental.pallas import tpu as pltpu
from jax.experimental.pallas import tpu_sc as plsc

NF = 100
NN = 120
NSEG = 128
B = 16384
D = 128
NCORES = 2
NSUB = 16
NW = NCORES * NSUB
ROWS = B // NW
CH = 128
NCHUNK = ROWS // CH


def _sc_segsum(reps_hbm, tgt_hbm, zsum_hbm, sums_out,
               rows_v, tgt_v, shared_sums, sems):
    cid = lax.axis_index("c")
    sid = lax.axis_index("s")
    wid = cid * NSUB + sid
    base = wid * ROWS

    pltpu.sync_copy(tgt_hbm.at[pl.ds(wid * NCHUNK, NCHUNK)], tgt_v)
    cps = [
        pltpu.async_copy(reps_hbm.at[pl.ds(base + j * CH, CH)],
                         rows_v.at[pl.ds(j * CH, CH)], sems[j])
        for j in range(NCHUNK)
    ]

    @pl.when(sid == 0)
    def _zero():
        pltpu.sync_copy(zsum_hbm, shared_sums)

    plsc.subcore_barrier()

    for j in range(NCHUNK):
        cps[j].wait()
        pltpu.sync_copy(rows_v.at[pl.ds(j * CH, CH)],
                        shared_sums.at[tgt_v.at[j]], add=True)

    plsc.subcore_barrier()

    @pl.when(sid == 0)
    def _writeout():
        pltpu.sync_copy(shared_sums, sums_out.at[cid])


@functools.lru_cache(maxsize=1)
def _make_seg_call():
    return pl.kernel(
        _sc_segsum,
        out_type=jax.ShapeDtypeStruct((NCORES, NSEG, D), jnp.float32),
        mesh=plsc.VectorSubcoreMesh(core_axis_name="c", subcore_axis_name="s"),
        scratch_types=[
            pltpu.VMEM((ROWS, D), jnp.float32),
            pltpu.VMEM((NCHUNK, CH), jnp.int32),
            pltpu.VMEM_SHARED((NSEG, D), jnp.float32),
            [pltpu.SemaphoreType.DMA] * NCHUNK,
        ],
    )


def _tc_counts(tgt_ref, o_ref):
    f32 = jnp.float32
    classrow = lax.broadcasted_iota(jnp.int32, (1, NSEG), 1)
    acc = jnp.zeros((1, NSEG), f32)
    for k in range(NW * NCHUNK):
        col = tgt_ref[:, k:k + 1]
        oh = jnp.where(col == classrow, 1.0, 0.0)
        acc = acc + jnp.sum(oh, axis=0, keepdims=True)
    o_ref[...] = jnp.broadcast_to(acc, (8, NSEG))


_counts_call = pl.pallas_call(
    _tc_counts,
    out_shape=jax.ShapeDtypeStruct((8, NSEG), jnp.float32),
)


def _tc_tail(s_ref, c_ref, lm_ref, td_ref, o_ref):
    f32 = jnp.float32
    sums = s_ref[0] + s_ref[1]
    cnt_row = c_ref[0:1, :]

    lm = lm_ref[0:1, :]
    ri = lax.broadcasted_iota(jnp.int32, (NSEG, NSEG), 0)
    ci = lax.broadcasted_iota(jnp.int32, (NSEG, NSEG), 1)
    eye = ri == ci

    a_fine = jnp.where(eye & (ri < NF), 1.0, 0.0)
    a_coarse = jnp.where((ri >= NF) & (ri < NN) & (lm == ri), 1.0, 0.0)
    amat = (a_fine + a_coarse).astype(f32)

    node_sums = jnp.dot(amat, sums, preferred_element_type=f32)
    node_cnt = jnp.sum(amat * cnt_row, axis=1, keepdims=True)

    present = node_cnt > 0.0
    safe = jnp.where(present, node_cnt, 1.0)
    means = node_sums / safe

    gram = lax.dot_general(means, means, (((1,), (1,)), ((), ())),
                           preferred_element_type=f32)
    diag_col = jnp.sum(jnp.where(eye, gram, 0.0), axis=1, keepdims=True)
    diag_row = jnp.sum(jnp.where(eye, gram, 0.0), axis=0, keepdims=True)
    d2 = jnp.maximum(diag_col + diag_row - 2.0 * gram, 0.0)
    dist = jnp.sqrt(d2)

    node_cnt_row = jnp.sum(jnp.where(eye, node_cnt * jnp.ones((1, NSEG), f32),
                                     0.0), axis=0, keepdims=True)
    w = jnp.where((node_cnt > 0.0) & (node_cnt_row > 0.0)
                  & (ri < ci) & (ci < NN), 1.0, 0.0).astype(f32)

    y = td_ref[...]
    m = jnp.sum(w)
    mx = jnp.sum(w * dist) / m
    my = jnp.sum(w * y) / m
    xc = w * (dist - mx)
    yc = w * (y - my)
    num = jnp.sum(xc * yc)
    den = jnp.sqrt(jnp.sum(xc * xc)) * jnp.sqrt(jnp.sum(yc * yc))
    res = 1.0 - num / den
    res = jnp.where(jnp.isnan(res), jnp.array(1.0, f32), res)
    o_ref[0, 0] = res


_tail_call = pl.pallas_call(
    _tc_tail,
    out_shape=jax.ShapeDtypeStruct((1, 1), jnp.float32),
    out_specs=pl.BlockSpec(memory_space=pltpu.SMEM),
)


def kernel(representations, targets_fine, label_map, tree_dist):
    tgt2d = targets_fine.reshape(NW * NCHUNK, CH)
    zsum = np.zeros((NSEG, D), np.float32)
    lm_pad = jnp.pad(label_map[:, 1], (0, NSEG - NF))[None, :]
    td_pad = jnp.pad(tree_dist, ((0, NSEG - NN), (0, NSEG - NN)))

    sums2 = _make_seg_call()(representations, tgt2d, zsum)
    cnts = _counts_call(tgt2d)
    outb = _tail_call(sums2, cnts, lm_pad, td_pad)
    return outb.reshape(())
